# baseline (speedup 1.0000x reference)
import functools
import numpy as np
import jax
import jax.numpy as jnp
from jax import lax
from jax.experimental import pallas as pl
from jax.experimental.pallas import tpu as pltpu, tpu_sc as plsc

BB, SS, DD, EE = 4, 8192, 768, 64
NN = BB * SS
CAP = int(NN * 1.1 / EE)
ZC = 0.001

BLK = 256
NBLK = NN // BLK

_KEY_BASE = 0x3C000000
_KEY_MAX = 0x03800000

NW = 16
TPW = NN // NW
NV = TPW // 16
BK = 64
HW = EE * BK
NPASS = 7

_PASS_TAB = np.zeros((8, 16), np.int32)
for _p, _r in enumerate([
    (26, 15, 20, 63, 0, 15, 0, 6, 63, 0, 0),
    (20, 15, 14, 63, 0, 15, 0, 6, 63, 0, 0),
    (14, 15, 8, 63, 0, 15, 0, 6, 63, 0, 0),
    (8, 15, 2, 63, 0, 15, 0, 6, 63, 0, 0),
    (2, 15, 0, 3, 4, 11, 15, 2, 3, 4, 15),
    (0, 11, 0, 0, 6, 5, 63, 0, 0, 6, 63),
    (0, 5, 0, 0, 5, 0, 31, 0, 0, 5, 31),
]):
    _PASS_TAB[_p, :len(_r)] = _r

_sc_mesh = plsc.VectorSubcoreMesh(core_axis_name="c", subcore_axis_name="s")

_I16 = lambda: lax.iota(jnp.int32, 16)


def _splat(x):
    return jnp.full((16,), x, jnp.int32)


def _sget(ref, flat_idx):
    return jnp.max(plsc.load_gather(ref, [_splat(flat_idx)]))


def _router_body(x_ref, w_ref, probs_ref, eidx_ref, key_ref, psum_ref, zsum_ref):
    i = pl.program_id(0)
    xb = x_ref[...]
    w = w_ref[...]
    logits = lax.dot_general(
        xb, w, (((1,), (1,)), ((), ())),
        preferred_element_type=jnp.float32)
    m = jnp.max(logits, axis=-1, keepdims=True)
    ex = jnp.exp(logits - m)
    s = jnp.sum(ex, axis=-1, keepdims=True)
    p = ex / s
    probs_ref[...] = p

    pmax = jnp.max(p, axis=-1, keepdims=True)
    lane = lax.broadcasted_iota(jnp.int32, (BLK, EE), 1)
    eid = jnp.min(jnp.where(p == pmax, lane, EE), axis=-1, keepdims=True)
    bits = lax.bitcast_convert_type(pmax, jnp.int32)
    key = jnp.clip(bits - _KEY_BASE, 0, _KEY_MAX)
    eidx_ref[...] = eid
    key_ref[...] = key

    lse = m + jnp.log(s)
    zpart = jnp.sum(lse * lse)
    ppart = jnp.sum(p, axis=0, keepdims=True)

    @pl.when(i == 0)
    def _init():
        psum_ref[...] = jnp.zeros_like(psum_ref)
        zsum_ref[...] = jnp.zeros_like(zsum_ref)

    psum_ref[...] += jnp.broadcast_to(ppart, psum_ref.shape)
    zsum_ref[...] += jnp.full(zsum_ref.shape, zpart, jnp.float32)


@functools.partial(
    pl.kernel, mesh=_sc_mesh,
    compiler_params=pltpu.CompilerParams(needs_layout_passes=False),
    out_type=[jax.ShapeDtypeStruct((EE, 16), jnp.int32)],
    scratch_types=[
        pltpu.VMEM((TPW,), jnp.int32),
        pltpu.VMEM((TPW,), jnp.int32),
        pltpu.VMEM((HW,), jnp.int32),
        pltpu.VMEM((4 * BK,), jnp.int32),
        pltpu.VMEM((4 * BK,), jnp.int32),
        pltpu.VMEM((EE * 16,), jnp.int32),
        pltpu.VMEM((16,), jnp.int32),
        pltpu.VMEM((128,), jnp.int32),
        pltpu.VMEM_SHARED((NW * HW,), jnp.int32),
        pltpu.VMEM_SHARED((EE * 16,), jnp.int32),
    ],
)
def _sc_select(key_hbm, eidx_hbm, tab_hbm, out_hbm, key_v, eidx_v, hist_v,
               hsum_v, slab_v, state_v, row_v, tab_v, gslab, gstate):
    cid = lax.axis_index("c")
    sid = lax.axis_index("s")

    @pl.when(cid == 0)
    def _():
        w = sid
        base = w * TPW
        pltpu.sync_copy(key_hbm.at[pl.ds(base, TPW)], key_v)
        pltpu.sync_copy(eidx_hbm.at[pl.ds(base, TPW)], eidx_v)
        pltpu.sync_copy(tab_hbm, tab_v)
        ones = jnp.ones((16,), jnp.int32)
        zeros = jnp.zeros((16,), jnp.int32)

        def zinit(i, _):
            state_v[pl.ds(i * 16, 16)] = zeros
            return 0
        lax.fori_loop(0, EE, zinit, 0)

        def one_pass(p, _):
            a_sh = _sget(tab_v, p * 16 + 0)
            ra_sh = _sget(tab_v, p * 16 + 1)
            dk_sh = _sget(tab_v, p * 16 + 2)
            dk_mask = _sget(tab_v, p * 16 + 3)
            drb = _sget(tab_v, p * 16 + 4)
            dr_sh = _sget(tab_v, p * 16 + 5)
            dr_mask = _sget(tab_v, p * 16 + 6)
            kb = _sget(tab_v, p * 16 + 7)
            kpm = _sget(tab_v, p * 16 + 8)
            rb = _sget(tab_v, p * 16 + 9)
            rmask = _sget(tab_v, p * 16 + 10)

            def zbody(i, _):
                hist_v[pl.ds(i * 16, 16)] = zeros
                return 0
            lax.fori_loop(0, HW // 16, zbody, 0)

            def tbody(i, _):
                k = key_v[pl.ds(i * 16, 16)]
                e = eidx_v[pl.ds(i * 16, 16)]
                rev = _splat(NN - 1 - base) - (_I16() + i * 16)
                pk = plsc.load_gather(state_v, [e * 16 + 0])
                pr = plsc.load_gather(state_v, [e * 16 + 1])
                act = ((k >> a_sh) == pk) & ((rev >> ra_sh) == pr)
                dig = (((k >> dk_sh) & dk_mask) << drb) | ((rev >> dr_sh) & dr_mask)
                plsc.addupdate_scatter(hist_v, [e * BK + dig], ones, mask=act)
                return 0
            lax.fori_loop(0, NV, tbody, 0)

            pltpu.sync_copy(hist_v, gslab.at[pl.ds(w * HW, HW)])
            plsc.subcore_barrier()

            myoff = (4 * w) * BK

            def cinit(i, _):
                hsum_v[pl.ds(i * 16, 16)] = zeros
                return 0
            lax.fori_loop(0, 4 * BK // 16, cinit, 0)

            def csrc(src, _):
                pltpu.sync_copy(
                    gslab.at[pl.ds(src * HW + myoff, 4 * BK)], slab_v)
                def cadd(i, _):
                    hsum_v[pl.ds(i * 16, 16)] += slab_v[pl.ds(i * 16, 16)]
                    return 0
                lax.fori_loop(0, 4 * BK // 16, cadd, 0)
                return 0
            lax.fori_loop(0, NW, csrc, 0)

            for j in range(4):
                e = 4 * w + j
                pk0 = _sget(state_v, e * 16 + 0)
                pr0 = _sget(state_v, e * 16 + 1)
                r0g = _sget(state_v, e * 16 + 2)
                ne0 = _sget(state_v, e * 16 + 3)
                r0 = jnp.where(p == 0, jnp.int32(CAP), r0g)

                def scan_v(v, carry):
                    best, above, tot = carry
                    vec = hsum_v[pl.ds(j * BK + (3 - v) * 16, 16)]
                    suf = lax.rev(plsc.cumsum(lax.rev(vec, (0,))), (0,)) + above
                    cand = jnp.max(
                        jnp.where(suf >= r0, _I16() + (3 - v) * 16, -1))
                    vtot = jnp.max(plsc.cumsum(vec))
                    return (jnp.maximum(best, cand), above + vtot, tot + vtot)
                best, _, tot = lax.fori_loop(
                    0, 4, scan_v, (jnp.int32(-1), jnp.int32(0), jnp.int32(0)))

                def gsum(v, acc):
                    vec = hsum_v[pl.ds(j * BK + v * 16, 16)]
                    gv = jnp.where((_I16() + v * 16) > best, vec, 0)
                    return acc + jnp.max(plsc.cumsum(gv))
                g = lax.fori_loop(0, 4, gsum, jnp.int32(0))

                ne = jnp.where(p == 0, tot, ne0)
                r1 = r0 - g
                t = best
                pk1 = (pk0 << kb) | ((t >> drb) & kpm)
                pr1 = (pr0 << rb) | (t & rmask)

                @pl.when(p == NPASS - 1)
                def _():
                    keep_all = ne <= CAP
                    tn = jnp.where(keep_all, jnp.int32(-1), pk1)
                    tr = jnp.where(keep_all, jnp.int32(0), pr1)
                    use = jnp.minimum(ne, CAP)
                    row_v[...] = (jnp.where(_I16() == 0, tn, 0)
                                  + jnp.where(_I16() == 1, tr, 0)
                                  + jnp.where(_I16() == 2, use, 0))
                    pltpu.sync_copy(row_v, out_hbm.at[e])

                @pl.when(p < NPASS - 1)
                def _():
                    row_v[...] = (jnp.where(_I16() == 0, pk1, 0)
                                  + jnp.where(_I16() == 1, pr1, 0)
                                  + jnp.where(_I16() == 2, r1, 0)
                                  + jnp.where(_I16() == 3, ne, 0))
                    pltpu.sync_copy(row_v, gstate.at[pl.ds(e * 16, 16)])

            plsc.subcore_barrier()

            @pl.when(p < NPASS - 1)
            def _():
                pltpu.sync_copy(gstate, state_v)
            plsc.subcore_barrier()
            return 0

        lax.fori_loop(0, NPASS, one_pass, 0)


def _dispatch_body(eidx_ref, key_ref, tho_ref, psum_ref, zsum_ref,
                   out_ref, aux_ref):
    i = pl.program_id(0)
    eid = eidx_ref[...]
    key = key_ref[...]
    tho = tho_ref[...]
    tn = tho[0:1, 0:EE]
    tr = tho[0:1, EE:2 * EE]
    lane = lax.broadcasted_iota(jnp.int32, (BLK, EE), 1)
    sub = lax.broadcasted_iota(jnp.int32, (BLK, 1), 0)
    rev = (NN - 1) - (i * BLK + sub)
    onehot = eid == lane
    keep = (key > tn) | ((key == tn) & (rev >= tr))
    out_ref[...] = (onehot & keep).astype(jnp.float32)

    @pl.when(i == 0)
    def _aux():
        use = tho[1:2, 0:EE].astype(jnp.float32)
        ps = psum_ref[0:1, :]
        lb = jnp.sum(ps * use)
        z = zsum_ref[0, 0]
        aux = (EE * lb / (NN * NN)) + ZC * (z / NN)
        aux_ref[...] = jnp.full(aux_ref.shape, aux, jnp.float32)


def kernel(x, W):
    x2 = x.reshape(NN, DD)

    probs, eidx_col, key_col, psum, zsum = pl.pallas_call(
        _router_body,
        grid=(NBLK,),
        in_specs=[
            pl.BlockSpec((BLK, DD), lambda i: (i, 0)),
            pl.BlockSpec((EE, DD), lambda i: (0, 0)),
        ],
        out_specs=[
            pl.BlockSpec((BLK, EE), lambda i: (i, 0)),
            pl.BlockSpec((BLK, 1), lambda i: (i, 0)),
            pl.BlockSpec((BLK, 1), lambda i: (i, 0)),
            pl.BlockSpec((8, EE), lambda i: (0, 0)),
            pl.BlockSpec((8, 64), lambda i: (0, 0)),
        ],
        out_shape=[
            jax.ShapeDtypeStruct((NN, EE), jnp.float32),
            jax.ShapeDtypeStruct((NN, 1), jnp.int32),
            jax.ShapeDtypeStruct((NN, 1), jnp.int32),
            jax.ShapeDtypeStruct((8, EE), jnp.float32),
            jax.ShapeDtypeStruct((8, 64), jnp.float32),
        ],
    )(x2, W)

    tab = jnp.asarray(_PASS_TAB.reshape(-1))
    (sel,) = _sc_select(key_col.reshape(NN), eidx_col.reshape(NN), tab)

    tn = sel[:, 0]
    tr = sel[:, 1]
    use = sel[:, 2]
    row0 = jnp.concatenate([tn, tr])[None, :]
    row1 = jnp.concatenate([use, jnp.zeros((EE,), jnp.int32)])[None, :]
    tho = jnp.concatenate(
        [row0, row1, jnp.zeros((6, 128), jnp.int32)], axis=0)

    disp, aux = pl.pallas_call(
        _dispatch_body,
        grid=(NBLK,),
        in_specs=[
            pl.BlockSpec((BLK, 1), lambda i: (i, 0)),
            pl.BlockSpec((BLK, 1), lambda i: (i, 0)),
            pl.BlockSpec((8, 128), lambda i: (0, 0)),
            pl.BlockSpec((8, EE), lambda i: (0, 0)),
            pl.BlockSpec((8, 64), lambda i: (0, 0)),
        ],
        out_specs=[
            pl.BlockSpec((BLK, EE), lambda i: (i, 0)),
            pl.BlockSpec((8, 64), lambda i: (0, 0)),
        ],
        out_shape=[
            jax.ShapeDtypeStruct((NN, EE), jnp.float32),
            jax.ShapeDtypeStruct((8, 64), jnp.float32),
        ],
    )(eidx_col, key_col, tho, psum, zsum)

    dispatch = disp.reshape(BB, SS, EE)
    router_probs = probs.reshape(BB, SS, EE)
    aux_loss = aux[0, 0]
    return (dispatch, dispatch, router_probs, aux_loss)


# speedup vs baseline: 1.4370x; 1.4370x over previous
import functools
import numpy as np
import jax
import jax.numpy as jnp
from jax import lax
from jax.experimental import pallas as pl
from jax.experimental.pallas import tpu as pltpu, tpu_sc as plsc

BB, SS, DD, EE = 4, 8192, 768, 64
NN = BB * SS
CAP = int(NN * 1.1 / EE)
ZC = 0.001

BLK = 512
NBLK = NN // BLK
CBLK = 2048
NCBLK = NN // CBLK

_KEY_BASE = 0x3C000000
_KEY_MAX = 0x03800000

NW = 16
TPW = NN // NW
NV = TPW // 16
BK = 64
HW = EE * BK
NPASS = 7

_PASS_TAB = np.zeros((8, 16), np.int32)
for _p, _r in enumerate([
    (26, 15, 20, 63, 0, 15, 0, 6, 63, 0, 0),
    (20, 15, 14, 63, 0, 15, 0, 6, 63, 0, 0),
    (14, 15, 8, 63, 0, 15, 0, 6, 63, 0, 0),
    (8, 15, 2, 63, 0, 15, 0, 6, 63, 0, 0),
    (2, 15, 0, 3, 4, 11, 15, 2, 3, 4, 15),
    (0, 11, 0, 0, 6, 5, 63, 0, 0, 6, 63),
    (0, 5, 0, 0, 5, 0, 31, 0, 0, 5, 31),
]):
    _PASS_TAB[_p, :len(_r)] = _r

_sc_mesh = plsc.VectorSubcoreMesh(core_axis_name="c", subcore_axis_name="s")

_I16 = lambda: lax.iota(jnp.int32, 16)


def _splat(x):
    return jnp.full((16,), x, jnp.int32)


def _sget(ref, flat_idx):
    return jnp.max(plsc.load_gather(ref, [_splat(flat_idx)]))


def _router_body(x_ref, w_ref, probs_ref, eidx_ref, key_ref, psum_ref, zsum_ref):
    i = pl.program_id(0)
    xb = x_ref[...]
    w = w_ref[...]
    logits = lax.dot_general(
        xb, w, (((1,), (1,)), ((), ())),
        preferred_element_type=jnp.float32)
    m = jnp.max(logits, axis=-1, keepdims=True)
    ex = jnp.exp(logits - m)
    s = jnp.sum(ex, axis=-1, keepdims=True)
    p = ex / s
    probs_ref[...] = p

    pmax = jnp.max(p, axis=-1, keepdims=True)
    lane = lax.broadcasted_iota(jnp.int32, (BLK, EE), 1)
    eid = jnp.min(jnp.where(p == pmax, lane, EE), axis=-1, keepdims=True)
    bits = lax.bitcast_convert_type(pmax, jnp.int32)
    key = jnp.clip(bits - _KEY_BASE, 0, _KEY_MAX)
    eidx_ref[...] = eid
    key_ref[...] = key

    lse = m + jnp.log(s)
    zpart = jnp.sum(lse * lse)
    ppart = jnp.sum(p, axis=0, keepdims=True)

    @pl.when(i == 0)
    def _init():
        psum_ref[...] = jnp.zeros_like(psum_ref)
        zsum_ref[...] = jnp.zeros_like(zsum_ref)

    psum_ref[...] += jnp.broadcast_to(ppart, psum_ref.shape)
    zsum_ref[...] += jnp.full(zsum_ref.shape, zpart, jnp.float32)


@functools.partial(
    pl.kernel, mesh=_sc_mesh,
    compiler_params=pltpu.CompilerParams(needs_layout_passes=False),
    out_type=[jax.ShapeDtypeStruct((EE, 16), jnp.int32)],
    scratch_types=[
        pltpu.VMEM((TPW,), jnp.int32),
        pltpu.VMEM((TPW,), jnp.int32),
        pltpu.VMEM((HW,), jnp.int32),
        pltpu.VMEM((4 * BK,), jnp.int32),
        pltpu.VMEM((4 * BK,), jnp.int32),
        pltpu.VMEM((EE * 16,), jnp.int32),
        pltpu.VMEM((16,), jnp.int32),
        pltpu.VMEM((128,), jnp.int32),
        pltpu.VMEM_SHARED((NW * HW,), jnp.int32),
        pltpu.VMEM_SHARED((EE * 16,), jnp.int32),
    ],
)
def _sc_select(key_hbm, eidx_hbm, tab_hbm, out_hbm, key_v, eidx_v, hist_v,
               hsum_v, slab_v, state_v, row_v, tab_v, gslab, gstate):
    cid = lax.axis_index("c")
    sid = lax.axis_index("s")

    @pl.when(cid == 0)
    def _():
        w = sid
        base = w * TPW
        pltpu.sync_copy(key_hbm.at[pl.ds(base, TPW)], key_v)
        pltpu.sync_copy(eidx_hbm.at[pl.ds(base, TPW)], eidx_v)
        pltpu.sync_copy(tab_hbm, tab_v)
        ones = jnp.ones((16,), jnp.int32)
        zeros = jnp.zeros((16,), jnp.int32)

        def zinit(i, _):
            state_v[pl.ds(i * 16, 16)] = zeros
            return 0
        lax.fori_loop(0, EE, zinit, 0)

        def one_pass(p, _):
            a_sh = _sget(tab_v, p * 16 + 0)
            ra_sh = _sget(tab_v, p * 16 + 1)
            dk_sh = _sget(tab_v, p * 16 + 2)
            dk_mask = _sget(tab_v, p * 16 + 3)
            drb = _sget(tab_v, p * 16 + 4)
            dr_sh = _sget(tab_v, p * 16 + 5)
            dr_mask = _sget(tab_v, p * 16 + 6)
            kb = _sget(tab_v, p * 16 + 7)
            kpm = _sget(tab_v, p * 16 + 8)
            rb = _sget(tab_v, p * 16 + 9)
            rmask = _sget(tab_v, p * 16 + 10)

            def zbody(i, _):
                hist_v[pl.ds(i * 16, 16)] = zeros
                return 0
            lax.fori_loop(0, HW // 16, zbody, 0)

            def tbody(i, _):
                k = key_v[pl.ds(i * 16, 16)]
                e = eidx_v[pl.ds(i * 16, 16)]
                rev = _splat(NN - 1 - base) - (_I16() + i * 16)
                pk = plsc.load_gather(state_v, [e * 16 + 0])
                pr = plsc.load_gather(state_v, [e * 16 + 1])
                act = ((k >> a_sh) == pk) & ((rev >> ra_sh) == pr)
                dig = (((k >> dk_sh) & dk_mask) << drb) | ((rev >> dr_sh) & dr_mask)
                plsc.addupdate_scatter(hist_v, [e * BK + dig], ones, mask=act)
                return 0
            lax.fori_loop(0, NV, tbody, 0)

            pltpu.sync_copy(hist_v, gslab.at[pl.ds(w * HW, HW)])
            plsc.subcore_barrier()

            myoff = (4 * w) * BK

            def cinit(i, _):
                hsum_v[pl.ds(i * 16, 16)] = zeros
                return 0
            lax.fori_loop(0, 4 * BK // 16, cinit, 0)

            def csrc(src, _):
                pltpu.sync_copy(
                    gslab.at[pl.ds(src * HW + myoff, 4 * BK)], slab_v)
                def cadd(i, _):
                    hsum_v[pl.ds(i * 16, 16)] += slab_v[pl.ds(i * 16, 16)]
                    return 0
                lax.fori_loop(0, 4 * BK // 16, cadd, 0)
                return 0
            lax.fori_loop(0, NW, csrc, 0)

            for j in range(4):
                e = 4 * w + j
                pk0 = _sget(state_v, e * 16 + 0)
                pr0 = _sget(state_v, e * 16 + 1)
                r0g = _sget(state_v, e * 16 + 2)
                ne0 = _sget(state_v, e * 16 + 3)
                r0 = jnp.where(p == 0, jnp.int32(CAP), r0g)

                def scan_v(v, carry):
                    best, above, tot = carry
                    vec = hsum_v[pl.ds(j * BK + (3 - v) * 16, 16)]
                    suf = lax.rev(plsc.cumsum(lax.rev(vec, (0,))), (0,)) + above
                    cand = jnp.max(
                        jnp.where(suf >= r0, _I16() + (3 - v) * 16, -1))
                    vtot = jnp.max(plsc.cumsum(vec))
                    return (jnp.maximum(best, cand), above + vtot, tot + vtot)
                best, _, tot = lax.fori_loop(
                    0, 4, scan_v, (jnp.int32(-1), jnp.int32(0), jnp.int32(0)))

                def gsum(v, acc):
                    vec = hsum_v[pl.ds(j * BK + v * 16, 16)]
                    gv = jnp.where((_I16() + v * 16) > best, vec, 0)
                    return acc + jnp.max(plsc.cumsum(gv))
                g = lax.fori_loop(0, 4, gsum, jnp.int32(0))

                ne = jnp.where(p == 0, tot, ne0)
                r1 = r0 - g
                t = best
                pk1 = (pk0 << kb) | ((t >> drb) & kpm)
                pr1 = (pr0 << rb) | (t & rmask)

                @pl.when(p == NPASS - 1)
                def _():
                    keep_all = ne <= CAP
                    tn = jnp.where(keep_all, jnp.int32(-1), pk1)
                    tr = jnp.where(keep_all, jnp.int32(0), pr1)
                    use = jnp.minimum(ne, CAP)
                    row_v[...] = (jnp.where(_I16() == 0, tn, 0)
                                  + jnp.where(_I16() == 1, tr, 0)
                                  + jnp.where(_I16() == 2, use, 0))
                    pltpu.sync_copy(row_v, out_hbm.at[e])

                @pl.when(p < NPASS - 1)
                def _():
                    row_v[...] = (jnp.where(_I16() == 0, pk1, 0)
                                  + jnp.where(_I16() == 1, pr1, 0)
                                  + jnp.where(_I16() == 2, r1, 0)
                                  + jnp.where(_I16() == 3, ne, 0))
                    pltpu.sync_copy(row_v, gstate.at[pl.ds(e * 16, 16)])

            plsc.subcore_barrier()

            @pl.when(p < NPASS - 1)
            def _():
                pltpu.sync_copy(gstate, state_v)
            plsc.subcore_barrier()
            return 0

        lax.fori_loop(0, NPASS, one_pass, 0)


def _dispatch_body(eidx_ref, key_ref, tho_ref, psum_ref, zsum_ref,
                   out_ref, aux_ref):
    i = pl.program_id(0)
    eid = eidx_ref[...]
    key = key_ref[...]
    tho = tho_ref[...]
    tn = tho[0:1, 0:EE]
    tr = tho[0:1, EE:2 * EE]
    lane = lax.broadcasted_iota(jnp.int32, (CBLK, EE), 1)
    sub = lax.broadcasted_iota(jnp.int32, (CBLK, 1), 0)
    rev = (NN - 1) - (i * CBLK + sub)
    onehot = eid == lane
    keep = (key > tn) | ((key == tn) & (rev >= tr))
    out_ref[...] = (onehot & keep).astype(jnp.float32)

    @pl.when(i == 0)
    def _aux():
        use = tho[1:2, 0:EE].astype(jnp.float32)
        ps = psum_ref[0:1, :]
        lb = jnp.sum(ps * use)
        z = zsum_ref[0, 0]
        aux = (EE * lb / (NN * NN)) + ZC * (z / NN)
        aux_ref[...] = jnp.full(aux_ref.shape, aux, jnp.float32)


def kernel(x, W):
    x2 = x.reshape(NN, DD)

    probs, eidx_col, key_col, psum, zsum = pl.pallas_call(
        _router_body,
        grid=(NBLK,),
        in_specs=[
            pl.BlockSpec((BLK, DD), lambda i: (i, 0)),
            pl.BlockSpec((EE, DD), lambda i: (0, 0)),
        ],
        out_specs=[
            pl.BlockSpec((BLK, EE), lambda i: (i, 0)),
            pl.BlockSpec((BLK, 1), lambda i: (i, 0)),
            pl.BlockSpec((BLK, 1), lambda i: (i, 0)),
            pl.BlockSpec((8, EE), lambda i: (0, 0)),
            pl.BlockSpec((8, 64), lambda i: (0, 0)),
        ],
        out_shape=[
            jax.ShapeDtypeStruct((NN, EE), jnp.float32),
            jax.ShapeDtypeStruct((NN, 1), jnp.int32),
            jax.ShapeDtypeStruct((NN, 1), jnp.int32),
            jax.ShapeDtypeStruct((8, EE), jnp.float32),
            jax.ShapeDtypeStruct((8, 64), jnp.float32),
        ],
    )(x2, W)

    tab = jnp.asarray(_PASS_TAB.reshape(-1))
    (sel,) = _sc_select(key_col.reshape(NN), eidx_col.reshape(NN), tab)

    tn = sel[:, 0]
    tr = sel[:, 1]
    use = sel[:, 2]
    row0 = jnp.concatenate([tn, tr])[None, :]
    row1 = jnp.concatenate([use, jnp.zeros((EE,), jnp.int32)])[None, :]
    tho = jnp.concatenate(
        [row0, row1, jnp.zeros((6, 128), jnp.int32)], axis=0)

    disp, aux = pl.pallas_call(
        _dispatch_body,
        grid=(NCBLK,),
        in_specs=[
            pl.BlockSpec((CBLK, 1), lambda i: (i, 0)),
            pl.BlockSpec((CBLK, 1), lambda i: (i, 0)),
            pl.BlockSpec((8, 128), lambda i: (0, 0)),
            pl.BlockSpec((8, EE), lambda i: (0, 0)),
            pl.BlockSpec((8, 64), lambda i: (0, 0)),
        ],
        out_specs=[
            pl.BlockSpec((CBLK, EE), lambda i: (i, 0)),
            pl.BlockSpec((8, 64), lambda i: (0, 0)),
        ],
        out_shape=[
            jax.ShapeDtypeStruct((NN, EE), jnp.float32),
            jax.ShapeDtypeStruct((8, 64), jnp.float32),
        ],
    )(eidx_col, key_col, tho, psum, zsum)

    dispatch = disp.reshape(BB, SS, EE)
    router_probs = probs.reshape(BB, SS, EE)
    aux_loss = aux[0, 0]
    return (dispatch, dispatch, router_probs, aux_loss)


# speedup vs baseline: 1.6163x; 1.1248x over previous
import functools
import numpy as np
import jax
import jax.numpy as jnp
from jax import lax
from jax.experimental import pallas as pl
from jax.experimental.pallas import tpu as pltpu, tpu_sc as plsc

BB, SS, DD, EE = 4, 8192, 768, 64
NN = BB * SS
CAP = int(NN * 1.1 / EE)
ZC = 0.001

BLK = 1024
NBLK = NN // BLK
CBLK = 2048
NCBLK = NN // CBLK

_KEY_BASE = 0x3C000000
_KEY_MAX = 0x03800000

NW = 16
TPW = NN // NW
NV = TPW // 16
BK = 64
HW = EE * BK
NPASS = 7

_PASS_TAB = np.zeros((8, 16), np.int32)
for _p, _r in enumerate([
    (26, 15, 20, 63, 0, 15, 0, 6, 63, 0, 0),
    (20, 15, 14, 63, 0, 15, 0, 6, 63, 0, 0),
    (14, 15, 8, 63, 0, 15, 0, 6, 63, 0, 0),
    (8, 15, 2, 63, 0, 15, 0, 6, 63, 0, 0),
    (2, 15, 0, 3, 4, 11, 15, 2, 3, 4, 15),
    (0, 11, 0, 0, 6, 5, 63, 0, 0, 6, 63),
    (0, 5, 0, 0, 5, 0, 31, 0, 0, 5, 31),
]):
    _PASS_TAB[_p, :len(_r)] = _r

_sc_mesh = plsc.VectorSubcoreMesh(core_axis_name="c", subcore_axis_name="s")

_I16 = lambda: lax.iota(jnp.int32, 16)


def _splat(x):
    return jnp.full((16,), x, jnp.int32)


def _sget(ref, flat_idx):
    return jnp.max(plsc.load_gather(ref, [_splat(flat_idx)]))


def _router_body(x_ref, w_ref, probs_ref, eidx_ref, key_ref, psum_ref, zsum_ref):
    i = pl.program_id(0)
    xb = x_ref[...]
    w = w_ref[...]
    logits = lax.dot_general(
        xb, w, (((1,), (1,)), ((), ())),
        preferred_element_type=jnp.float32)
    m = jnp.max(logits, axis=-1, keepdims=True)
    ex = jnp.exp(logits - m)
    s = jnp.sum(ex, axis=-1, keepdims=True)
    p = ex / s
    probs_ref[...] = p

    pmax = 1.0 / s
    lane = lax.broadcasted_iota(jnp.int32, (BLK, EE), 1)
    eid = jnp.min(jnp.where(p == pmax, lane, EE), axis=-1, keepdims=True)
    bits = lax.bitcast_convert_type(pmax, jnp.int32)
    key = jnp.clip(bits - _KEY_BASE, 0, _KEY_MAX)
    eidx_ref[...] = eid
    key_ref[...] = key

    lse = m + jnp.log(s)
    zpart = jnp.sum(lse * lse)
    ppart = jnp.sum(p, axis=0, keepdims=True)

    @pl.when(i == 0)
    def _init():
        psum_ref[...] = jnp.zeros_like(psum_ref)
        zsum_ref[...] = jnp.zeros_like(zsum_ref)

    psum_ref[...] += jnp.broadcast_to(ppart, psum_ref.shape)
    zsum_ref[...] += jnp.full(zsum_ref.shape, zpart, jnp.float32)


@functools.partial(
    pl.kernel, mesh=_sc_mesh,
    compiler_params=pltpu.CompilerParams(needs_layout_passes=False),
    out_type=[jax.ShapeDtypeStruct((EE, 16), jnp.int32)],
    scratch_types=[
        pltpu.VMEM((TPW,), jnp.int32),
        pltpu.VMEM((TPW,), jnp.int32),
        pltpu.VMEM((HW,), jnp.int32),
        pltpu.VMEM((4 * BK,), jnp.int32),
        pltpu.VMEM((4 * BK,), jnp.int32),
        pltpu.VMEM((EE * 16,), jnp.int32),
        pltpu.VMEM((16,), jnp.int32),
        pltpu.VMEM((128,), jnp.int32),
        pltpu.VMEM_SHARED((NW * HW,), jnp.int32),
        pltpu.VMEM_SHARED((EE * 16,), jnp.int32),
    ],
)
def _sc_select(key_hbm, eidx_hbm, tab_hbm, out_hbm, key_v, eidx_v, hist_v,
               hsum_v, slab_v, state_v, row_v, tab_v, gslab, gstate):
    cid = lax.axis_index("c")
    sid = lax.axis_index("s")

    @pl.when(cid == 0)
    def _():
        w = sid
        base = w * TPW
        pltpu.sync_copy(key_hbm.at[pl.ds(base, TPW)], key_v)
        pltpu.sync_copy(eidx_hbm.at[pl.ds(base, TPW)], eidx_v)
        pltpu.sync_copy(tab_hbm, tab_v)
        ones = jnp.ones((16,), jnp.int32)
        zeros = jnp.zeros((16,), jnp.int32)

        def zinit(i, _):
            state_v[pl.ds(i * 16, 16)] = zeros
            return 0
        lax.fori_loop(0, EE, zinit, 0)

        def one_pass(p, _):
            a_sh = _sget(tab_v, p * 16 + 0)
            ra_sh = _sget(tab_v, p * 16 + 1)
            dk_sh = _sget(tab_v, p * 16 + 2)
            dk_mask = _sget(tab_v, p * 16 + 3)
            drb = _sget(tab_v, p * 16 + 4)
            dr_sh = _sget(tab_v, p * 16 + 5)
            dr_mask = _sget(tab_v, p * 16 + 6)
            kb = _sget(tab_v, p * 16 + 7)
            kpm = _sget(tab_v, p * 16 + 8)
            rb = _sget(tab_v, p * 16 + 9)
            rmask = _sget(tab_v, p * 16 + 10)

            def zbody(i, _):
                hist_v[pl.ds(i * 16, 16)] = zeros
                return 0
            lax.fori_loop(0, HW // 16, zbody, 0)

            def tbody(i, _):
                k = key_v[pl.ds(i * 16, 16)]
                e = eidx_v[pl.ds(i * 16, 16)]
                rev = _splat(NN - 1 - base) - (_I16() + i * 16)
                pk = plsc.load_gather(state_v, [e * 16 + 0])
                pr = plsc.load_gather(state_v, [e * 16 + 1])
                act = ((k >> a_sh) == pk) & ((rev >> ra_sh) == pr)
                dig = (((k >> dk_sh) & dk_mask) << drb) | ((rev >> dr_sh) & dr_mask)
                plsc.addupdate_scatter(hist_v, [e * BK + dig], ones, mask=act)
                return 0
            lax.fori_loop(0, NV, tbody, 0)

            pltpu.sync_copy(hist_v, gslab.at[pl.ds(w * HW, HW)])
            plsc.subcore_barrier()

            myoff = (4 * w) * BK

            def cinit(i, _):
                hsum_v[pl.ds(i * 16, 16)] = zeros
                return 0
            lax.fori_loop(0, 4 * BK // 16, cinit, 0)

            def csrc(src, _):
                pltpu.sync_copy(
                    gslab.at[pl.ds(src * HW + myoff, 4 * BK)], slab_v)
                def cadd(i, _):
                    hsum_v[pl.ds(i * 16, 16)] += slab_v[pl.ds(i * 16, 16)]
                    return 0
                lax.fori_loop(0, 4 * BK // 16, cadd, 0)
                return 0
            lax.fori_loop(0, NW, csrc, 0)

            for j in range(4):
                e = 4 * w + j
                pk0 = _sget(state_v, e * 16 + 0)
                pr0 = _sget(state_v, e * 16 + 1)
                r0g = _sget(state_v, e * 16 + 2)
                ne0 = _sget(state_v, e * 16 + 3)
                r0 = jnp.where(p == 0, jnp.int32(CAP), r0g)

                def scan_v(v, carry):
                    best, above, tot = carry
                    vec = hsum_v[pl.ds(j * BK + (3 - v) * 16, 16)]
                    suf = lax.rev(plsc.cumsum(lax.rev(vec, (0,))), (0,)) + above
                    cand = jnp.max(
                        jnp.where(suf >= r0, _I16() + (3 - v) * 16, -1))
                    vtot = jnp.max(plsc.cumsum(vec))
                    return (jnp.maximum(best, cand), above + vtot, tot + vtot)
                best, _, tot = lax.fori_loop(
                    0, 4, scan_v, (jnp.int32(-1), jnp.int32(0), jnp.int32(0)))

                def gsum(v, acc):
                    vec = hsum_v[pl.ds(j * BK + v * 16, 16)]
                    gv = jnp.where((_I16() + v * 16) > best, vec, 0)
                    return acc + jnp.max(plsc.cumsum(gv))
                g = lax.fori_loop(0, 4, gsum, jnp.int32(0))

                ne = jnp.where(p == 0, tot, ne0)
                r1 = r0 - g
                t = best
                pk1 = (pk0 << kb) | ((t >> drb) & kpm)
                pr1 = (pr0 << rb) | (t & rmask)

                @pl.when(p == NPASS - 1)
                def _():
                    keep_all = ne <= CAP
                    tn = jnp.where(keep_all, jnp.int32(-1), pk1)
                    tr = jnp.where(keep_all, jnp.int32(0), pr1)
                    use = jnp.minimum(ne, CAP)
                    row_v[...] = (jnp.where(_I16() == 0, tn, 0)
                                  + jnp.where(_I16() == 1, tr, 0)
                                  + jnp.where(_I16() == 2, use, 0))
                    pltpu.sync_copy(row_v, out_hbm.at[e])

                @pl.when(p < NPASS - 1)
                def _():
                    row_v[...] = (jnp.where(_I16() == 0, pk1, 0)
                                  + jnp.where(_I16() == 1, pr1, 0)
                                  + jnp.where(_I16() == 2, r1, 0)
                                  + jnp.where(_I16() == 3, ne, 0))
                    pltpu.sync_copy(row_v, gstate.at[pl.ds(e * 16, 16)])

            plsc.subcore_barrier()

            @pl.when(p < NPASS - 1)
            def _():
                pltpu.sync_copy(gstate, state_v)
            plsc.subcore_barrier()
            return 0

        lax.fori_loop(0, NPASS, one_pass, 0)


def _dispatch_body(eidx_ref, key_ref, tho_ref, psum_ref, zsum_ref,
                   out_ref, aux_ref):
    i = pl.program_id(0)
    eid = eidx_ref[...]
    key = key_ref[...]
    tho = tho_ref[...]
    tn = tho[0:1, 0:EE]
    tr = tho[0:1, EE:2 * EE]
    lane = lax.broadcasted_iota(jnp.int32, (CBLK, EE), 1)
    sub = lax.broadcasted_iota(jnp.int32, (CBLK, 1), 0)
    rev = (NN - 1) - (i * CBLK + sub)
    onehot = eid == lane
    keep = (key > tn) | ((key == tn) & (rev >= tr))
    out_ref[...] = (onehot & keep).astype(jnp.float32)

    @pl.when(i == 0)
    def _aux():
        use = tho[1:2, 0:EE].astype(jnp.float32)
        ps = psum_ref[0:1, :]
        lb = jnp.sum(ps * use)
        z = zsum_ref[0, 0]
        aux = (EE * lb / (NN * NN)) + ZC * (z / NN)
        aux_ref[...] = jnp.full(aux_ref.shape, aux, jnp.float32)


def kernel(x, W):
    x2 = x.reshape(NN, DD)

    probs, eidx_col, key_col, psum, zsum = pl.pallas_call(
        _router_body,
        grid=(NBLK,),
        in_specs=[
            pl.BlockSpec((BLK, DD), lambda i: (i, 0)),
            pl.BlockSpec((EE, DD), lambda i: (0, 0)),
        ],
        out_specs=[
            pl.BlockSpec((BLK, EE), lambda i: (i, 0)),
            pl.BlockSpec((BLK, 1), lambda i: (i, 0)),
            pl.BlockSpec((BLK, 1), lambda i: (i, 0)),
            pl.BlockSpec((8, EE), lambda i: (0, 0)),
            pl.BlockSpec((8, 64), lambda i: (0, 0)),
        ],
        out_shape=[
            jax.ShapeDtypeStruct((NN, EE), jnp.float32),
            jax.ShapeDtypeStruct((NN, 1), jnp.int32),
            jax.ShapeDtypeStruct((NN, 1), jnp.int32),
            jax.ShapeDtypeStruct((8, EE), jnp.float32),
            jax.ShapeDtypeStruct((8, 64), jnp.float32),
        ],
    )(x2, W)

    tab = jnp.asarray(_PASS_TAB.reshape(-1))
    (sel,) = _sc_select(key_col.reshape(NN), eidx_col.reshape(NN), tab)

    tn = sel[:, 0]
    tr = sel[:, 1]
    use = sel[:, 2]
    row0 = jnp.concatenate([tn, tr])[None, :]
    row1 = jnp.concatenate([use, jnp.zeros((EE,), jnp.int32)])[None, :]
    tho = jnp.concatenate(
        [row0, row1, jnp.zeros((6, 128), jnp.int32)], axis=0)

    disp, aux = pl.pallas_call(
        _dispatch_body,
        grid=(NCBLK,),
        in_specs=[
            pl.BlockSpec((CBLK, 1), lambda i: (i, 0)),
            pl.BlockSpec((CBLK, 1), lambda i: (i, 0)),
            pl.BlockSpec((8, 128), lambda i: (0, 0)),
            pl.BlockSpec((8, EE), lambda i: (0, 0)),
            pl.BlockSpec((8, 64), lambda i: (0, 0)),
        ],
        out_specs=[
            pl.BlockSpec((CBLK, EE), lambda i: (i, 0)),
            pl.BlockSpec((8, 64), lambda i: (0, 0)),
        ],
        out_shape=[
            jax.ShapeDtypeStruct((NN, EE), jnp.float32),
            jax.ShapeDtypeStruct((8, 64), jnp.float32),
        ],
    )(eidx_col, key_col, tho, psum, zsum)

    dispatch = disp.reshape(BB, SS, EE)
    router_probs = probs.reshape(BB, SS, EE)
    aux_loss = aux[0, 0]
    return (dispatch, dispatch, router_probs, aux_loss)


# speedup vs baseline: 1.7503x; 1.0829x over previous
import functools
import numpy as np
import jax
import jax.numpy as jnp
from jax import lax
from jax.experimental import pallas as pl
from jax.experimental.pallas import tpu as pltpu, tpu_sc as plsc

BB, SS, DD, EE = 4, 8192, 768, 64
NN = BB * SS
CAP = int(NN * 1.1 / EE)
ZC = 0.001

BLK = 1024
NBLK = NN // BLK
CBLK = 2048
NCBLK = NN // CBLK

_KEY_BASE = 0x3C000000
_KEY_MAX = 0x03800000

NW = 16
TPW = NN // NW
NV = TPW // 16
BK = 64
HW = EE * BK
NPASS = 7

_PASS_TAB = np.zeros((8, 16), np.int32)
for _p, _r in enumerate([
    (26, 15, 20, 63, 0, 15, 0, 6, 63, 0, 0),
    (20, 15, 14, 63, 0, 15, 0, 6, 63, 0, 0),
    (14, 15, 8, 63, 0, 15, 0, 6, 63, 0, 0),
    (8, 15, 2, 63, 0, 15, 0, 6, 63, 0, 0),
    (2, 15, 0, 3, 4, 11, 15, 2, 3, 4, 15),
    (0, 11, 0, 0, 6, 5, 63, 0, 0, 6, 63),
    (0, 5, 0, 0, 5, 0, 31, 0, 0, 5, 31),
]):
    _PASS_TAB[_p, :len(_r)] = _r

_sc_mesh = plsc.VectorSubcoreMesh(core_axis_name="c", subcore_axis_name="s")

_I16 = lambda: lax.iota(jnp.int32, 16)


def _splat(x):
    return jnp.full((16,), x, jnp.int32)


def _sget(ref, flat_idx):
    return jnp.max(plsc.load_gather(ref, [_splat(flat_idx)]))


def _router_body(x_ref, w_ref, probs_ref, pk_ref, psum_ref, zsum_ref):
    i = pl.program_id(0)
    xb = x_ref[...]
    w = w_ref[...]
    logits = lax.dot_general(
        xb, w, (((1,), (1,)), ((), ())),
        preferred_element_type=jnp.float32)
    m = jnp.max(logits, axis=-1, keepdims=True)
    ex = jnp.exp(logits - m)
    s = jnp.sum(ex, axis=-1, keepdims=True)
    p = ex / s
    probs_ref[...] = p

    pmax = 1.0 / s
    lane = lax.broadcasted_iota(jnp.int32, (BLK, EE), 1)
    eid = jnp.min(jnp.where(p == pmax, lane, EE), axis=-1, keepdims=True)
    bits = lax.bitcast_convert_type(pmax, jnp.int32)
    key = jnp.clip(bits - _KEY_BASE, 0, _KEY_MAX)
    pk_ref[...] = (key << 6) | eid

    lse = m + jnp.log(s)
    zpart = jnp.sum(lse * lse)
    ppart = jnp.sum(p, axis=0, keepdims=True)

    @pl.when(i == 0)
    def _init():
        psum_ref[...] = jnp.zeros_like(psum_ref)
        zsum_ref[...] = jnp.zeros_like(zsum_ref)

    psum_ref[...] += jnp.broadcast_to(ppart, psum_ref.shape)
    zsum_ref[...] += jnp.full(zsum_ref.shape, zpart, jnp.float32)


@functools.partial(
    pl.kernel, mesh=_sc_mesh,
    compiler_params=pltpu.CompilerParams(needs_layout_passes=False),
    out_type=[jax.ShapeDtypeStruct((EE, 16), jnp.int32)],
    scratch_types=[
        pltpu.VMEM((TPW,), jnp.int32),
        pltpu.VMEM((HW,), jnp.int32),
        pltpu.VMEM((4 * BK,), jnp.int32),
        pltpu.VMEM((4 * BK,), jnp.int32),
        pltpu.VMEM((EE * 16,), jnp.int32),
        pltpu.VMEM((16,), jnp.int32),
        pltpu.VMEM((128,), jnp.int32),
        pltpu.VMEM_SHARED((NW * HW,), jnp.int32),
        pltpu.VMEM_SHARED((EE * 16,), jnp.int32),
    ],
)
def _sc_select(pk_hbm, tab_hbm, out_hbm, pk_v, hist_v,
               hsum_v, slab_v, state_v, row_v, tab_v, gslab, gstate):
    cid = lax.axis_index("c")
    sid = lax.axis_index("s")

    @pl.when(cid == 0)
    def _():
        w = sid
        base = w * TPW
        pltpu.sync_copy(pk_hbm.at[pl.ds(base, TPW)], pk_v)
        pltpu.sync_copy(tab_hbm, tab_v)
        ones = jnp.ones((16,), jnp.int32)
        zeros = jnp.zeros((16,), jnp.int32)

        def zinit(i, _):
            state_v[pl.ds(i * 16, 16)] = zeros
            return 0
        lax.fori_loop(0, EE, zinit, 0)

        def one_pass(p, _):
            a_sh = _sget(tab_v, p * 16 + 0)
            ra_sh = _sget(tab_v, p * 16 + 1)
            dk_sh = _sget(tab_v, p * 16 + 2)
            dk_mask = _sget(tab_v, p * 16 + 3)
            drb = _sget(tab_v, p * 16 + 4)
            dr_sh = _sget(tab_v, p * 16 + 5)
            dr_mask = _sget(tab_v, p * 16 + 6)
            kb = _sget(tab_v, p * 16 + 7)
            kpm = _sget(tab_v, p * 16 + 8)
            rb = _sget(tab_v, p * 16 + 9)
            rmask = _sget(tab_v, p * 16 + 10)

            def zbody(i, _):
                hist_v[pl.ds(i * 16, 16)] = zeros
                return 0
            lax.fori_loop(0, HW // 16, zbody, 0)

            def tbody(i, _):
                pv = pk_v[pl.ds(i * 16, 16)]
                k = lax.shift_right_logical(pv, 6)
                e = pv & 63
                rev = _splat(NN - 1 - base) - (_I16() + i * 16)
                pk = plsc.load_gather(state_v, [e * 16 + 0])
                pr = plsc.load_gather(state_v, [e * 16 + 1])
                act = ((k >> a_sh) == pk) & ((rev >> ra_sh) == pr)
                dig = (((k >> dk_sh) & dk_mask) << drb) | ((rev >> dr_sh) & dr_mask)
                plsc.addupdate_scatter(hist_v, [e * BK + dig], ones, mask=act)
                return 0
            lax.fori_loop(0, NV, tbody, 0)

            pltpu.sync_copy(hist_v, gslab.at[pl.ds(w * HW, HW)])
            plsc.subcore_barrier()

            myoff = (4 * w) * BK

            def cinit(i, _):
                hsum_v[pl.ds(i * 16, 16)] = zeros
                return 0
            lax.fori_loop(0, 4 * BK // 16, cinit, 0)

            def csrc(src, _):
                pltpu.sync_copy(
                    gslab.at[pl.ds(src * HW + myoff, 4 * BK)], slab_v)
                def cadd(i, _):
                    hsum_v[pl.ds(i * 16, 16)] += slab_v[pl.ds(i * 16, 16)]
                    return 0
                lax.fori_loop(0, 4 * BK // 16, cadd, 0)
                return 0
            lax.fori_loop(0, NW, csrc, 0)

            for j in range(4):
                e = 4 * w + j
                pk0 = _sget(state_v, e * 16 + 0)
                pr0 = _sget(state_v, e * 16 + 1)
                r0g = _sget(state_v, e * 16 + 2)
                ne0 = _sget(state_v, e * 16 + 3)
                r0 = jnp.where(p == 0, jnp.int32(CAP), r0g)

                def scan_v(v, carry):
                    best, above, tot = carry
                    vec = hsum_v[pl.ds(j * BK + (3 - v) * 16, 16)]
                    suf = lax.rev(plsc.cumsum(lax.rev(vec, (0,))), (0,)) + above
                    cand = jnp.max(
                        jnp.where(suf >= r0, _I16() + (3 - v) * 16, -1))
                    vtot = jnp.max(plsc.cumsum(vec))
                    return (jnp.maximum(best, cand), above + vtot, tot + vtot)
                best, _, tot = lax.fori_loop(
                    0, 4, scan_v, (jnp.int32(-1), jnp.int32(0), jnp.int32(0)))

                def gsum(v, acc):
                    vec = hsum_v[pl.ds(j * BK + v * 16, 16)]
                    gv = jnp.where((_I16() + v * 16) > best, vec, 0)
                    return acc + jnp.max(plsc.cumsum(gv))
                g = lax.fori_loop(0, 4, gsum, jnp.int32(0))

                ne = jnp.where(p == 0, tot, ne0)
                r1 = r0 - g
                t = best
                pk1 = (pk0 << kb) | ((t >> drb) & kpm)
                pr1 = (pr0 << rb) | (t & rmask)

                @pl.when(p == NPASS - 1)
                def _():
                    keep_all = ne <= CAP
                    tn = jnp.where(keep_all, jnp.int32(-1), pk1)
                    tr = jnp.where(keep_all, jnp.int32(0), pr1)
                    use = jnp.minimum(ne, CAP)
                    row_v[...] = (jnp.where(_I16() == 0, tn, 0)
                                  + jnp.where(_I16() == 1, tr, 0)
                                  + jnp.where(_I16() == 2, use, 0))
                    pltpu.sync_copy(row_v, out_hbm.at[e])

                @pl.when(p < NPASS - 1)
                def _():
                    row_v[...] = (jnp.where(_I16() == 0, pk1, 0)
                                  + jnp.where(_I16() == 1, pr1, 0)
                                  + jnp.where(_I16() == 2, r1, 0)
                                  + jnp.where(_I16() == 3, ne, 0))
                    pltpu.sync_copy(row_v, gstate.at[pl.ds(e * 16, 16)])

            plsc.subcore_barrier()

            @pl.when(p < NPASS - 1)
            def _():
                pltpu.sync_copy(gstate, state_v)
            plsc.subcore_barrier()
            return 0

        lax.fori_loop(0, NPASS, one_pass, 0)


def _dispatch_body(pk_ref, tho_ref, psum_ref, zsum_ref,
                   out_ref, aux_ref):
    i = pl.program_id(0)
    packed = pk_ref[...]
    eid = packed & 63
    key = lax.shift_right_logical(packed, 6)
    tho = tho_ref[...]
    tn = tho[0:1, 0:EE]
    tr = tho[0:1, EE:2 * EE]
    lane = lax.broadcasted_iota(jnp.int32, (CBLK, EE), 1)
    sub = lax.broadcasted_iota(jnp.int32, (CBLK, 1), 0)
    rev = (NN - 1) - (i * CBLK + sub)
    onehot = eid == lane
    keep = (key > tn) | ((key == tn) & (rev >= tr))
    out_ref[...] = (onehot & keep).astype(jnp.float32)

    @pl.when(i == 0)
    def _aux():
        use = tho[1:2, 0:EE].astype(jnp.float32)
        ps = psum_ref[0:1, :]
        lb = jnp.sum(ps * use)
        z = zsum_ref[0, 0]
        aux = (EE * lb / (NN * NN)) + ZC * (z / NN)
        aux_ref[...] = jnp.full(aux_ref.shape, aux, jnp.float32)


def kernel(x, W):
    x2 = x.reshape(NN, DD)

    probs, packed_col, psum, zsum = pl.pallas_call(
        _router_body,
        grid=(NBLK,),
        in_specs=[
            pl.BlockSpec((BLK, DD), lambda i: (i, 0)),
            pl.BlockSpec((EE, DD), lambda i: (0, 0)),
        ],
        out_specs=[
            pl.BlockSpec((BLK, EE), lambda i: (i, 0)),
            pl.BlockSpec((BLK, 1), lambda i: (i, 0)),
            pl.BlockSpec((8, EE), lambda i: (0, 0)),
            pl.BlockSpec((8, 64), lambda i: (0, 0)),
        ],
        out_shape=[
            jax.ShapeDtypeStruct((NN, EE), jnp.float32),
            jax.ShapeDtypeStruct((NN, 1), jnp.int32),
            jax.ShapeDtypeStruct((8, EE), jnp.float32),
            jax.ShapeDtypeStruct((8, 64), jnp.float32),
        ],
    )(x2, W)

    tab = jnp.asarray(_PASS_TAB.reshape(-1))
    (sel,) = _sc_select(packed_col.reshape(NN), tab)

    tn = sel[:, 0]
    tr = sel[:, 1]
    use = sel[:, 2]
    row0 = jnp.concatenate([tn, tr])[None, :]
    row1 = jnp.concatenate([use, jnp.zeros((EE,), jnp.int32)])[None, :]
    tho = jnp.concatenate(
        [row0, row1, jnp.zeros((6, 128), jnp.int32)], axis=0)

    disp, aux = pl.pallas_call(
        _dispatch_body,
        grid=(NCBLK,),
        in_specs=[
            pl.BlockSpec((CBLK, 1), lambda i: (i, 0)),
            pl.BlockSpec((8, 128), lambda i: (0, 0)),
            pl.BlockSpec((8, EE), lambda i: (0, 0)),
            pl.BlockSpec((8, 64), lambda i: (0, 0)),
        ],
        out_specs=[
            pl.BlockSpec((CBLK, EE), lambda i: (i, 0)),
            pl.BlockSpec((8, 64), lambda i: (0, 0)),
        ],
        out_shape=[
            jax.ShapeDtypeStruct((NN, EE), jnp.float32),
            jax.ShapeDtypeStruct((8, 64), jnp.float32),
        ],
    )(packed_col, tho, psum, zsum)

    dispatch = disp.reshape(BB, SS, EE)
    router_probs = probs.reshape(BB, SS, EE)
    aux_loss = aux[0, 0]
    return (dispatch, dispatch, router_probs, aux_loss)
